# baseline (speedup 1.0000x reference)
import functools

import jax
import jax.numpy as jnp
from jax import lax
from jax.experimental import pallas as pl
from jax.experimental.pallas import tpu as pltpu

_DELTA_PULL = 0.5
_DELTA_PUSH = 1.5
_W_PULL = 1.0
_W_PUSH = 1.0
_W_REG = 0.001
_EPS = 1e-12
_K = 48


def _body(emb_ref, lab_ref, lp_ref, lq_ref, lr_ref, kp_ref,
          sums_s, counts_s, cl_s, *, nt):
    ph = pl.program_id(1)
    t = pl.program_id(2)

    x = emb_ref[0]
    lbl = lab_ref[0, 0]
    tt = x.shape[1]
    kiota = lax.broadcasted_iota(jnp.int32, (_K, tt), 0)
    oh = (lbl[None, :] == kiota).astype(jnp.float32)

    @pl.when(ph == 0)
    def _phase0():
        @pl.when(t == 0)
        def _init():
            sums_s[...] = jnp.zeros_like(sums_s)
            counts_s[...] = jnp.zeros_like(counts_s)

        sums_s[...] += lax.dot_general(
            oh, x, (((1,), (1,)), ((), ())),
            preferred_element_type=jnp.float32)
        counts_s[...] += lax.dot_general(
            oh, jnp.ones((1, tt), jnp.float32), (((1,), (1,)), ((), ())),
            preferred_element_type=jnp.float32)

    @pl.when(ph == 1)
    def _phase1():
        @pl.when(t == 0)
        def _init():
            cl_s[...] = jnp.zeros_like(cl_s)

        counts_c = jnp.maximum(counts_s[...], 1.0)
        centers = sums_s[...] / counts_c
        dots = lax.dot_general(
            centers, x, (((1,), (0,)), ((), ())),
            preferred_element_type=jnp.float32)
        cn2 = jnp.sum(centers * centers, axis=1, keepdims=True)
        sel = jnp.sum(oh * (dots - 0.5 * cn2), axis=0)
        en2 = jnp.sum(x * x, axis=0)
        d2 = jnp.maximum(en2 - 2.0 * sel, 0.0) + _EPS
        dist = jnp.sqrt(d2)
        hinged = jnp.where(lbl > 0,
                           jnp.maximum(dist - _DELTA_PULL, 0.0) ** 2,
                           0.0)
        cl_s[...] += lax.dot_general(
            oh, hinged.reshape(tt, 1), (((1,), (0,)), ((), ())),
            preferred_element_type=jnp.float32)

        @pl.when(t == nt - 1)
        def _finalize():
            counts_raw = counts_s[...]
            counts_cc = jnp.maximum(counts_raw, 1.0)
            cen = sums_s[...] / counts_cc
            kidx = lax.broadcasted_iota(jnp.int32, (_K, 1), 0)
            pf = jnp.where((counts_raw > 0.0) & (kidx >= 1), 1.0, 0.0)
            kp = jnp.sum(pf)
            kf = jnp.maximum(kp, 1.0)
            cen2 = jnp.sum(cen * cen, axis=1, keepdims=True)
            l_pull = jnp.sum(pf * (cl_s[...] / counts_cc)) / kf
            norms = jnp.sqrt(cen2 + _EPS)
            l_reg = jnp.sum(pf * norms) / kf
            gram = lax.dot_general(
                cen, cen, (((1,), (1,)), ((), ())),
                preferred_element_type=jnp.float32)
            cn2_row = lax.dot_general(
                jnp.ones((1, cen.shape[1]), jnp.float32), cen * cen,
                (((1,), (1,)), ((), ())),
                preferred_element_type=jnp.float32)
            pw2 = jnp.maximum(cen2 + cn2_row - 2.0 * gram, 0.0)
            pw = jnp.sqrt(pw2 + _EPS)
            ii = lax.broadcasted_iota(jnp.int32, (_K, _K), 0)
            jj = lax.broadcasted_iota(jnp.int32, (_K, _K), 1)
            pair_f = lax.dot_general(
                pf, pf, (((1,), (1,)), ((), ())),
                preferred_element_type=jnp.float32)
            pair_f = pair_f * jnp.where(jj > ii, 1.0, 0.0)
            hv = pair_f * jnp.maximum(2.0 * _DELTA_PUSH - pw, 0.0) ** 2
            npairs = jnp.sum(pair_f)
            l_push = jnp.where(npairs > 0.0,
                               jnp.sum(hv) / jnp.maximum(npairs, 1.0),
                               0.0)
            lp_ref[...] = jnp.reshape(l_pull, (1, 1, 1))
            lq_ref[...] = jnp.reshape(l_push, (1, 1, 1))
            lr_ref[...] = jnp.reshape(l_reg, (1, 1, 1))
            kp_ref[...] = jnp.reshape(kp, (1, 1, 1))


def kernel(embedding, ins_label):
    b, e = embedding.shape[0], embedding.shape[1]
    n = embedding.shape[2] * embedding.shape[3]
    t = 8192 if n % 8192 == 0 else n
    nt = n // t
    emb = embedding.reshape(b, e, n)
    lab = ins_label.reshape(b * nt, 1, t)

    out_shape = [jax.ShapeDtypeStruct((b, 1, 1), jnp.float32)] * 4
    out_spec = pl.BlockSpec((1, 1, 1), lambda bi, ph, ti: (bi, 0, 0))
    lp, lq, lr, kp = pl.pallas_call(
        functools.partial(_body, nt=nt),
        grid=(b, 2, nt),
        in_specs=[
            pl.BlockSpec((1, e, t), lambda bi, ph, ti: (bi, 0, ti)),
            pl.BlockSpec((1, 1, t), lambda bi, ph, ti: (bi * nt + ti, 0, 0)),
        ],
        out_specs=[out_spec] * 4,
        out_shape=out_shape,
        scratch_shapes=[
            pltpu.VMEM((_K, e), jnp.float32),
            pltpu.VMEM((_K, 1), jnp.float32),
            pltpu.VMEM((_K, 1), jnp.float32),
        ],
    )(emb, lab)

    lp = lp.reshape(b)
    lq = lq.reshape(b)
    lr = lr.reshape(b)
    kp = kp.reshape(b)
    has = (kp > 0.0).astype(jnp.float32)
    nvalid = jnp.maximum(jnp.sum(has), 1.0)
    l_pull = jnp.sum(has * lp) / nvalid
    l_push = jnp.sum(has * lq) / nvalid
    l_reg = jnp.sum(has * lr) / nvalid
    total = _W_PULL * l_pull + _W_PUSH * l_push + _W_REG * l_reg
    return {"loss": total, "l_pull": l_pull, "l_push": l_push,
            "l_reg": l_reg}


# speedup vs baseline: 1.2361x; 1.2361x over previous
import functools

import jax
import jax.numpy as jnp
from jax import lax
from jax.experimental import pallas as pl
from jax.experimental.pallas import tpu as pltpu

_DELTA_PULL = 0.5
_DELTA_PUSH = 1.5
_W_PULL = 1.0
_W_PUSH = 1.0
_W_REG = 0.001
_EPS = 1e-12
_K = 48


def _body(emb_ref, lab_ref, lp_ref, lq_ref, lr_ref, kp_ref,
          sums_s, counts_s, cl_s, *, nt):
    ph = pl.program_id(1)
    t = pl.program_id(2)

    x = emb_ref[0]
    lbl = lab_ref[0, 0]
    tt = x.shape[1]
    kiota = lax.broadcasted_iota(jnp.int32, (_K, tt), 0)
    oh = (lbl[None, :] == kiota).astype(jnp.float32)

    @pl.when(ph == 0)
    def _phase0():
        @pl.when(t == 0)
        def _init():
            sums_s[...] = jnp.zeros_like(sums_s)
            counts_s[...] = jnp.zeros_like(counts_s)

        sums_s[...] += lax.dot_general(
            oh, x, (((1,), (1,)), ((), ())),
            preferred_element_type=jnp.float32)
        counts_s[...] += jnp.sum(oh, axis=1, keepdims=True)

    @pl.when(ph == 1)
    def _phase1():
        @pl.when(t == 0)
        def _init():
            cl_s[...] = jnp.zeros_like(cl_s)

        counts_c = jnp.maximum(counts_s[...], 1.0)
        centers = sums_s[...] / counts_c
        dots = lax.dot_general(
            centers, x, (((1,), (0,)), ((), ())),
            preferred_element_type=jnp.float32)
        cn2 = jnp.sum(centers * centers, axis=1, keepdims=True)
        sel = jnp.sum(oh * (dots - 0.5 * cn2), axis=0)
        en2 = jnp.sum(x * x, axis=0)
        d2 = jnp.maximum(en2 - 2.0 * sel, 0.0) + _EPS
        dist = jnp.sqrt(d2)
        hinged = jnp.where(lbl > 0,
                           jnp.maximum(dist - _DELTA_PULL, 0.0) ** 2,
                           0.0)
        cl_s[...] += jnp.sum(oh * hinged[None, :], axis=1, keepdims=True)

        @pl.when(t == nt - 1)
        def _finalize():
            counts_raw = counts_s[...]
            counts_cc = jnp.maximum(counts_raw, 1.0)
            cen = sums_s[...] / counts_cc
            kidx = lax.broadcasted_iota(jnp.int32, (_K, 1), 0)
            pf = jnp.where((counts_raw > 0.0) & (kidx >= 1), 1.0, 0.0)
            kp = jnp.sum(pf)
            kf = jnp.maximum(kp, 1.0)
            cen2 = jnp.sum(cen * cen, axis=1, keepdims=True)
            l_pull = jnp.sum(pf * (cl_s[...] / counts_cc)) / kf
            norms = jnp.sqrt(cen2 + _EPS)
            l_reg = jnp.sum(pf * norms) / kf
            gram = lax.dot_general(
                cen, cen, (((1,), (1,)), ((), ())),
                preferred_element_type=jnp.float32)
            cn2_row = lax.dot_general(
                jnp.ones((1, cen.shape[1]), jnp.float32), cen * cen,
                (((1,), (1,)), ((), ())),
                preferred_element_type=jnp.float32)
            pw2 = jnp.maximum(cen2 + cn2_row - 2.0 * gram, 0.0)
            pw = jnp.sqrt(pw2 + _EPS)
            ii = lax.broadcasted_iota(jnp.int32, (_K, _K), 0)
            jj = lax.broadcasted_iota(jnp.int32, (_K, _K), 1)
            pair_f = lax.dot_general(
                pf, pf, (((1,), (1,)), ((), ())),
                preferred_element_type=jnp.float32)
            pair_f = pair_f * jnp.where(jj > ii, 1.0, 0.0)
            hv = pair_f * jnp.maximum(2.0 * _DELTA_PUSH - pw, 0.0) ** 2
            npairs = jnp.sum(pair_f)
            l_push = jnp.where(npairs > 0.0,
                               jnp.sum(hv) / jnp.maximum(npairs, 1.0),
                               0.0)
            lp_ref[...] = jnp.reshape(l_pull, (1, 1, 1))
            lq_ref[...] = jnp.reshape(l_push, (1, 1, 1))
            lr_ref[...] = jnp.reshape(l_reg, (1, 1, 1))
            kp_ref[...] = jnp.reshape(kp, (1, 1, 1))


def kernel(embedding, ins_label):
    b, e = embedding.shape[0], embedding.shape[1]
    n = embedding.shape[2] * embedding.shape[3]
    t = 8192 if n % 8192 == 0 else n
    nt = n // t
    emb = embedding.reshape(b, e, n)
    lab = ins_label.reshape(b * nt, 1, t)

    out_shape = [jax.ShapeDtypeStruct((b, 1, 1), jnp.float32)] * 4
    out_spec = pl.BlockSpec((1, 1, 1), lambda bi, ph, ti: (bi, 0, 0))
    lp, lq, lr, kp = pl.pallas_call(
        functools.partial(_body, nt=nt),
        grid=(b, 2, nt),
        in_specs=[
            pl.BlockSpec((1, e, t), lambda bi, ph, ti: (bi, 0, ti)),
            pl.BlockSpec((1, 1, t), lambda bi, ph, ti: (bi * nt + ti, 0, 0)),
        ],
        out_specs=[out_spec] * 4,
        out_shape=out_shape,
        scratch_shapes=[
            pltpu.VMEM((_K, e), jnp.float32),
            pltpu.VMEM((_K, 1), jnp.float32),
            pltpu.VMEM((_K, 1), jnp.float32),
        ],
    )(emb, lab)

    lp = lp.reshape(b)
    lq = lq.reshape(b)
    lr = lr.reshape(b)
    kp = kp.reshape(b)
    has = (kp > 0.0).astype(jnp.float32)
    nvalid = jnp.maximum(jnp.sum(has), 1.0)
    l_pull = jnp.sum(has * lp) / nvalid
    l_push = jnp.sum(has * lq) / nvalid
    l_reg = jnp.sum(has * lr) / nvalid
    total = _W_PULL * l_pull + _W_PUSH * l_push + _W_REG * l_reg
    return {"loss": total, "l_pull": l_pull, "l_push": l_push,
            "l_reg": l_reg}


# speedup vs baseline: 1.2845x; 1.0392x over previous
import functools

import jax
import jax.numpy as jnp
from jax import lax
from jax.experimental import pallas as pl
from jax.experimental.pallas import tpu as pltpu

_DELTA_PULL = 0.5
_DELTA_PUSH = 1.5
_W_PULL = 1.0
_W_PUSH = 1.0
_W_REG = 0.001
_EPS = 1e-12
_K = 48


def _body(emb_ref, lab_ref, lp_ref, lq_ref, lr_ref, kp_ref,
          sums_s, counts_s, cl_s, *, nt):
    ph = pl.program_id(1)
    t = pl.program_id(2)

    tt = emb_ref.shape[2]
    cc = 512 if tt % 512 == 0 else tt
    nck = tt // cc
    kiota = lax.broadcasted_iota(jnp.int32, (_K, cc), 0)

    @pl.when(ph == 0)
    def _phase0():
        @pl.when(t == 0)
        def _init():
            sums_s[...] = jnp.zeros_like(sums_s)
            counts_s[...] = jnp.zeros_like(counts_s)

        for c in range(nck):
            x = emb_ref[0, :, c * cc:(c + 1) * cc]
            lbl = lab_ref[0, 0, c * cc:(c + 1) * cc]
            oh = (lbl[None, :] == kiota).astype(jnp.float32)
            sums_s[...] += lax.dot_general(
                oh, x, (((1,), (1,)), ((), ())),
                preferred_element_type=jnp.float32)
            counts_s[...] += jnp.sum(oh, axis=1, keepdims=True)

    @pl.when(ph == 1)
    def _phase1():
        @pl.when(t == 0)
        def _init():
            cl_s[...] = jnp.zeros_like(cl_s)

        counts_c = jnp.maximum(counts_s[...], 1.0)
        centers = sums_s[...] / counts_c
        cn2 = jnp.sum(centers * centers, axis=1, keepdims=True)
        for c in range(nck):
            x = emb_ref[0, :, c * cc:(c + 1) * cc]
            lbl = lab_ref[0, 0, c * cc:(c + 1) * cc]
            oh = (lbl[None, :] == kiota).astype(jnp.float32)
            dots = lax.dot_general(
                centers, x, (((1,), (0,)), ((), ())),
                preferred_element_type=jnp.float32)
            sel = jnp.sum(oh * (dots - 0.5 * cn2), axis=0)
            en2 = jnp.sum(x * x, axis=0)
            d2 = jnp.maximum(en2 - 2.0 * sel, 0.0) + _EPS
            dist = jnp.sqrt(d2)
            hinged = jnp.where(lbl > 0,
                               jnp.maximum(dist - _DELTA_PULL, 0.0) ** 2,
                               0.0)
            cl_s[...] += jnp.sum(oh * hinged[None, :], axis=1,
                                 keepdims=True)

        @pl.when(t == nt - 1)
        def _finalize():
            counts_raw = counts_s[...]
            counts_cc = jnp.maximum(counts_raw, 1.0)
            cen = sums_s[...] / counts_cc
            kidx = lax.broadcasted_iota(jnp.int32, (_K, 1), 0)
            pf = jnp.where((counts_raw > 0.0) & (kidx >= 1), 1.0, 0.0)
            kp = jnp.sum(pf)
            kf = jnp.maximum(kp, 1.0)
            cen2 = jnp.sum(cen * cen, axis=1, keepdims=True)
            l_pull = jnp.sum(pf * (cl_s[...] / counts_cc)) / kf
            norms = jnp.sqrt(cen2 + _EPS)
            l_reg = jnp.sum(pf * norms) / kf
            gram = lax.dot_general(
                cen, cen, (((1,), (1,)), ((), ())),
                preferred_element_type=jnp.float32)
            cn2_row = lax.dot_general(
                jnp.ones((1, cen.shape[1]), jnp.float32), cen * cen,
                (((1,), (1,)), ((), ())),
                preferred_element_type=jnp.float32)
            pw2 = jnp.maximum(cen2 + cn2_row - 2.0 * gram, 0.0)
            pw = jnp.sqrt(pw2 + _EPS)
            ii = lax.broadcasted_iota(jnp.int32, (_K, _K), 0)
            jj = lax.broadcasted_iota(jnp.int32, (_K, _K), 1)
            pair_f = lax.dot_general(
                pf, pf, (((1,), (1,)), ((), ())),
                preferred_element_type=jnp.float32)
            pair_f = pair_f * jnp.where(jj > ii, 1.0, 0.0)
            hv = pair_f * jnp.maximum(2.0 * _DELTA_PUSH - pw, 0.0) ** 2
            npairs = jnp.sum(pair_f)
            l_push = jnp.where(npairs > 0.0,
                               jnp.sum(hv) / jnp.maximum(npairs, 1.0),
                               0.0)
            lp_ref[...] = jnp.reshape(l_pull, (1, 1, 1))
            lq_ref[...] = jnp.reshape(l_push, (1, 1, 1))
            lr_ref[...] = jnp.reshape(l_reg, (1, 1, 1))
            kp_ref[...] = jnp.reshape(kp, (1, 1, 1))


def kernel(embedding, ins_label):
    b, e = embedding.shape[0], embedding.shape[1]
    n = embedding.shape[2] * embedding.shape[3]
    t = 8192 if n % 8192 == 0 else n
    nt = n // t
    emb = embedding.reshape(b, e, n)
    lab = ins_label.reshape(b * nt, 1, t)

    out_shape = [jax.ShapeDtypeStruct((b, 1, 1), jnp.float32)] * 4
    out_spec = pl.BlockSpec((1, 1, 1), lambda bi, ph, ti: (bi, 0, 0))
    lp, lq, lr, kp = pl.pallas_call(
        functools.partial(_body, nt=nt),
        grid=(b, 2, nt),
        in_specs=[
            pl.BlockSpec((1, e, t), lambda bi, ph, ti: (bi, 0, ti)),
            pl.BlockSpec((1, 1, t), lambda bi, ph, ti: (bi * nt + ti, 0, 0)),
        ],
        out_specs=[out_spec] * 4,
        out_shape=out_shape,
        scratch_shapes=[
            pltpu.VMEM((_K, e), jnp.float32),
            pltpu.VMEM((_K, 1), jnp.float32),
            pltpu.VMEM((_K, 1), jnp.float32),
        ],
    )(emb, lab)

    lp = lp.reshape(b)
    lq = lq.reshape(b)
    lr = lr.reshape(b)
    kp = kp.reshape(b)
    has = (kp > 0.0).astype(jnp.float32)
    nvalid = jnp.maximum(jnp.sum(has), 1.0)
    l_pull = jnp.sum(has * lp) / nvalid
    l_push = jnp.sum(has * lq) / nvalid
    l_reg = jnp.sum(has * lr) / nvalid
    total = _W_PULL * l_pull + _W_PUSH * l_push + _W_REG * l_reg
    return {"loss": total, "l_pull": l_pull, "l_push": l_push,
            "l_reg": l_reg}
